# baseline (speedup 1.0000x reference)
import functools

import jax
import jax.numpy as jnp
from jax import lax
from jax.experimental import pallas as pl
from jax.experimental.pallas import tpu as pltpu
from jax.experimental.pallas import tpu_sc as plsc

_NC = 2
_NS = 16
_NW = _NC * _NS
_G = 8
_K = 4


@functools.cache
def _build(batch: int, seq: int, dim: int):
    b_per_w = batch // _NW
    ngroup = b_per_w // _G
    mesh = plsc.VectorSubcoreMesh(core_axis_name="c", subcore_axis_name="s")

    @functools.partial(
        pl.kernel,
        mesh=mesh,
        out_type=jax.ShapeDtypeStruct((batch, seq, dim), jnp.float32),
        compiler_params=pltpu.CompilerParams(use_tc_tiling_on_sc=False),
        scratch_types=[
            pltpu.VMEM((b_per_w, seq), jnp.int32),
            pltpu.VMEM((_K, _G, seq, dim), jnp.float32),
            pltpu.SemaphoreType.DMA((_K,)),
            pltpu.SemaphoreType.DMA((_K,)),
        ],
    )
    def gather_kernel(idx_hbm, table_hbm, out_hbm, idx_v, rows_v, gsem, wsem):
        wid = lax.axis_index("s") * _NC + lax.axis_index("c")
        base = wid * b_per_w
        pltpu.sync_copy(idx_hbm.at[pl.ds(base, b_per_w)], idx_v)

        def start_group(g):
            b = g % _K
            return [
                pltpu.async_copy(
                    table_hbm.at[idx_v.at[g * _G + j]],
                    rows_v.at[b, j],
                    gsem.at[b],
                )
                for j in range(_G)
            ]

        def start_write(g):
            b = g % _K
            return pltpu.async_copy(
                rows_v.at[b],
                out_hbm.at[pl.ds(base + g * _G, _G)],
                wsem.at[b],
            )

        gathers = [None] * ngroup
        writes = [None] * ngroup
        waited = [False] * ngroup

        nprime = min(_K - 1, ngroup)
        for g in range(nprime):
            gathers[g] = start_group(g)

        for g in range(ngroup):
            nxt = g + _K - 1
            if nprime <= nxt < ngroup:
                prev = nxt - _K
                if prev >= 0:
                    writes[prev].wait()
                    waited[prev] = True
                gathers[nxt] = start_group(nxt)
            for c in gathers[g]:
                c.wait()
            writes[g] = start_write(g)

        for g in range(ngroup):
            if not waited[g]:
                writes[g].wait()

    return gather_kernel


def kernel(X, E_weight):
    batch, seq = X.shape
    dim = E_weight.shape[1]
    return _build(batch, seq, dim)(X.astype(jnp.int32), E_weight)
